# Initial kernel scaffold; baseline (speedup 1.0000x reference)
import jax
import jax.numpy as jnp
from jax.experimental import pallas as pl


def kernel(inputs, table, W1, b1, W2, b2):
    raise NotImplementedError("write your pallas kernel here")


# speedup vs baseline: 1.8724x; 1.8724x over previous
import functools

import jax
import jax.numpy as jnp
from jax import lax
from jax.experimental import pallas as pl
from jax.experimental.pallas import tpu as pltpu
from jax.experimental.pallas import tpu_sc as plsc

NC, NS = 2, 16
NW = NC * NS
B, S, E = 16384, 200, 64
HID, NCLS = 128, 1000
G0 = 128
G1 = S - G0
CH = 32
B_PER_W = B // NW


def _sc_embed_sum(inputs, table):
    mesh = plsc.VectorSubcoreMesh(core_axis_name="c", subcore_axis_name="s")
    flat = inputs.reshape(B * S)

    @functools.partial(
        pl.kernel,
        out_type=jax.ShapeDtypeStruct((B, E), jnp.float32),
        mesh=mesh,
        scratch_types=[
            pltpu.VMEM((CH * S,), jnp.int32),
            pltpu.VMEM((S, E), jnp.float32),
            pltpu.VMEM((CH, E), jnp.float32),
            pltpu.SemaphoreType.DMA,
        ],
        compiler_params=pltpu.CompilerParams(use_tc_tiling_on_sc=False),
    )
    def k(flat_hbm, table_hbm, out_hbm, idx_v, rows_v, out_v, sem):
        wid = lax.axis_index("s") * NC + lax.axis_index("c")
        base = wid * B_PER_W

        @pl.loop(0, B_PER_W, step=CH)
        def _(r0):
            off = pl.multiple_of((base + r0) * S, 8)
            pltpu.sync_copy(flat_hbm.at[pl.ds(off, CH * S)], idx_v)

            @pl.loop(0, CH)
            def _(i):
                o0 = pl.multiple_of(i * S, 8)
                o1 = pl.multiple_of(i * S + G0, 8)
                c0 = pltpu.async_copy(
                    table_hbm.at[idx_v.at[pl.ds(o0, G0)]],
                    rows_v.at[pl.ds(0, G0)], sem)
                c1 = pltpu.async_copy(
                    table_hbm.at[idx_v.at[pl.ds(o1, G1)]],
                    rows_v.at[pl.ds(G0, G1)], sem)
                c0.wait()
                c1.wait()

                z = jnp.zeros((16,), jnp.float32)

                def body(r, acc):
                    return tuple(
                        acc[j] + rows_v[r, 16 * j:16 * (j + 1)]
                        for j in range(4))

                acc = lax.fori_loop(0, S, body, (z, z, z, z))
                for j in range(4):
                    out_v[i, 16 * j:16 * (j + 1)] = acc[j]

            pltpu.sync_copy(out_v, out_hbm.at[pl.ds(base + r0, CH)])

    return k(flat, table)


def _tc_mlp(summed, W1, b1, W2, b2):
    BLK = 1024

    def body(x_ref, w1_ref, b1_ref, w2_ref, b2_ref, o_ref):
        x = jnp.tanh(x_ref[...])
        h = lax.dot_general(
            x, w1_ref[...], (((1,), (1,)), ((), ())),
            preferred_element_type=jnp.float32,
            precision=lax.Precision.HIGHEST)
        h = jnp.tanh(h + b1_ref[...])
        o = lax.dot_general(
            h, w2_ref[...], (((1,), (1,)), ((), ())),
            preferred_element_type=jnp.float32,
            precision=lax.Precision.HIGHEST)
        o_ref[...] = jnp.tanh(o + b2_ref[...])

    return pl.pallas_call(
        body,
        grid=(B // BLK,),
        in_specs=[
            pl.BlockSpec((BLK, E), lambda i: (i, 0)),
            pl.BlockSpec((HID, E), lambda i: (0, 0)),
            pl.BlockSpec((1, HID), lambda i: (0, 0)),
            pl.BlockSpec((NCLS, HID), lambda i: (0, 0)),
            pl.BlockSpec((1, NCLS), lambda i: (0, 0)),
        ],
        out_specs=pl.BlockSpec((BLK, NCLS), lambda i: (i, 0)),
        out_shape=jax.ShapeDtypeStruct((B, NCLS), jnp.float32),
    )(summed, W1, b1.reshape(1, HID), W2, b2.reshape(1, NCLS))


def kernel(inputs, table, W1, b1, W2, b2):
    summed = _sc_embed_sum(inputs, table)
    return _tc_mlp(summed, W1, b1, W2, b2)


# speedup vs baseline: 2.8550x; 1.5248x over previous
import functools

import jax
import jax.numpy as jnp
from jax import lax
from jax.experimental import pallas as pl
from jax.experimental.pallas import tpu as pltpu
from jax.experimental.pallas import tpu_sc as plsc

NC, NS = 2, 16
NW = NC * NS
B, S, E = 16384, 200, 64
HID, NCLS = 128, 1000
G0 = 128
G1 = S - G0
CH = 32
B_PER_W = B // NW


def _sc_embed_sum(inputs, table):
    mesh = plsc.VectorSubcoreMesh(core_axis_name="c", subcore_axis_name="s")

    @functools.partial(
        pl.kernel,
        out_type=jax.ShapeDtypeStruct((B, E), jnp.float32),
        mesh=mesh,
        scratch_types=[
            pltpu.VMEM((CH, S), jnp.int32),
            pltpu.VMEM((S, E), jnp.float32),
            pltpu.VMEM((S, E), jnp.float32),
            pltpu.VMEM((CH, E), jnp.float32),
            pltpu.SemaphoreType.DMA,
            pltpu.SemaphoreType.DMA,
        ],
        compiler_params=pltpu.CompilerParams(use_tc_tiling_on_sc=False),
    )
    def k(inputs_hbm, table_hbm, out_hbm, idx_v, rows0, rows1, out_v,
          sem0, sem1):
        wid = lax.axis_index("s") * NC + lax.axis_index("c")
        base = wid * B_PER_W
        bufs = (rows0, rows1)
        sems = (sem0, sem1)

        def issue(i, buf, sem):
            pltpu.async_copy(
                table_hbm.at[idx_v.at[i, pl.ds(0, G0)]],
                buf.at[pl.ds(0, G0)], sem)
            pltpu.async_copy(
                table_hbm.at[idx_v.at[i, pl.ds(G0, G1)]],
                buf.at[pl.ds(G0, G1)], sem)

        def drain(buf, sem):
            pltpu.make_async_copy(table_hbm.at[pl.ds(0, S)], buf, sem).wait()

        def accum(buf, i):
            z = jnp.zeros((16,), jnp.float32)

            def body(r, acc):
                return tuple(
                    acc[j] + buf[r, 16 * j:16 * (j + 1)] for j in range(4))

            acc = lax.fori_loop(0, S, body, (z, z, z, z), unroll=4)
            for j in range(4):
                out_v[i, 16 * j:16 * (j + 1)] = acc[j]

        @pl.loop(0, B_PER_W, step=CH)
        def _(r0):
            pltpu.sync_copy(inputs_hbm.at[pl.ds(base + r0, CH)], idx_v)
            issue(0, rows0, sem0)
            issue(1, rows1, sem1)

            @pl.loop(0, CH, step=2)
            def _(i):
                for b in range(2):
                    drain(bufs[b], sems[b])

                    @pl.when(i + 2 + b < CH)
                    def _():
                        issue(i + 2 + b, bufs[b], sems[b])

                    accum(bufs[b], i + b)

            pltpu.sync_copy(out_v, out_hbm.at[pl.ds(base + r0, CH)])

    return k(inputs, table)


def _tc_mlp(summed, W1, b1, W2, b2):
    BLK = 1024

    def body(x_ref, w1_ref, b1_ref, w2_ref, b2_ref, o_ref):
        x = jnp.tanh(x_ref[...])
        h = lax.dot_general(
            x, w1_ref[...], (((1,), (1,)), ((), ())),
            preferred_element_type=jnp.float32,
            precision=lax.Precision.HIGHEST)
        h = jnp.tanh(h + b1_ref[...])
        o = lax.dot_general(
            h, w2_ref[...], (((1,), (1,)), ((), ())),
            preferred_element_type=jnp.float32,
            precision=lax.Precision.HIGHEST)
        o_ref[...] = jnp.tanh(o + b2_ref[...])

    return pl.pallas_call(
        body,
        grid=(B // BLK,),
        in_specs=[
            pl.BlockSpec((BLK, E), lambda i: (i, 0)),
            pl.BlockSpec((HID, E), lambda i: (0, 0)),
            pl.BlockSpec((1, HID), lambda i: (0, 0)),
            pl.BlockSpec((NCLS, HID), lambda i: (0, 0)),
            pl.BlockSpec((1, NCLS), lambda i: (0, 0)),
        ],
        out_specs=pl.BlockSpec((BLK, NCLS), lambda i: (i, 0)),
        out_shape=jax.ShapeDtypeStruct((B, NCLS), jnp.float32),
    )(summed, W1, b1.reshape(1, HID), W2, b2.reshape(1, NCLS))


def kernel(inputs, table, W1, b1, W2, b2):
    summed = _sc_embed_sum(inputs, table)
    return _tc_mlp(summed, W1, b1, W2, b2)


# speedup vs baseline: 2.8743x; 1.0067x over previous
import functools

import jax
import jax.numpy as jnp
from jax import lax
from jax.experimental import pallas as pl
from jax.experimental.pallas import tpu as pltpu
from jax.experimental.pallas import tpu_sc as plsc

NC, NS = 2, 16
NW = NC * NS
B, S, E = 16384, 200, 64
HID, NCLS = 128, 1000
G0 = 128
G1 = S - G0
CH = 32
B_PER_W = B // NW


def _tc_flatten(inputs):
    FR = 64

    def body(x_ref, o_ref):
        for g in range(FR // 8):
            o_ref[16 * g:16 * g + 8, :] = x_ref[8 * g:8 * g + 8, 0:128]
            o_ref[16 * g + 8:16 * g + 16, 0:G1] = \
                x_ref[8 * g:8 * g + 8, G0:S]
            o_ref[16 * g + 8:16 * g + 16, G1:128] = \
                jnp.zeros((8, 128 - G1), jnp.int32)

    return pl.pallas_call(
        body,
        grid=(B // FR,),
        in_specs=[pl.BlockSpec((FR, S), lambda i: (i, 0))],
        out_specs=pl.BlockSpec((2 * FR, 128), lambda i: (i, 0)),
        out_shape=jax.ShapeDtypeStruct((2 * B, 128), jnp.int32),
    )(inputs)


def _sc_embed_sum(flat, table):
    mesh = plsc.VectorSubcoreMesh(core_axis_name="c", subcore_axis_name="s")

    @functools.partial(
        pl.kernel,
        out_type=jax.ShapeDtypeStruct((B, E), jnp.float32),
        mesh=mesh,
        scratch_types=[
            pltpu.VMEM((2 * CH, 128), jnp.int32),
            pltpu.VMEM((S, E), jnp.float32),
            pltpu.VMEM((S, E), jnp.float32),
            pltpu.VMEM((CH, E), jnp.float32),
            pltpu.SemaphoreType.DMA,
            pltpu.SemaphoreType.DMA,
        ],
        compiler_params=pltpu.CompilerParams(use_tc_tiling_on_sc=False),
    )
    def k(flat_hbm, table_hbm, out_hbm, idx_v, rows0, rows1, out_v,
          sem0, sem1):
        wid = lax.axis_index("s") * NC + lax.axis_index("c")
        base = wid * B_PER_W
        bufs = (rows0, rows1)
        sems = (sem0, sem1)

        def issue(i, buf, sem):
            row_a = 16 * (i // 8) + lax.rem(i, 8)
            pltpu.async_copy(
                table_hbm.at[idx_v.at[row_a]],
                buf.at[pl.ds(0, G0)], sem)
            pltpu.async_copy(
                table_hbm.at[idx_v.at[row_a + 8, pl.ds(0, G1)]],
                buf.at[pl.ds(G0, G1)], sem)

        def drain(buf, sem):
            pltpu.make_async_copy(table_hbm.at[pl.ds(0, S)], buf, sem).wait()

        def accum(buf, i):
            z = jnp.zeros((16,), jnp.float32)

            def body(r, acc):
                return tuple(
                    acc[j] + buf[r, 16 * j:16 * (j + 1)] for j in range(4))

            acc = lax.fori_loop(0, S, body, (z, z, z, z), unroll=4)
            for j in range(4):
                out_v[i, 16 * j:16 * (j + 1)] = acc[j]

        @pl.loop(0, B_PER_W, step=CH)
        def _(r0):
            pltpu.sync_copy(
                flat_hbm.at[pl.ds(2 * (base + r0), 2 * CH)], idx_v)
            issue(0, rows0, sem0)
            issue(1, rows1, sem1)

            @pl.loop(0, CH, step=2)
            def _(i):
                for b in range(2):
                    drain(bufs[b], sems[b])

                    @pl.when(i + 2 + b < CH)
                    def _():
                        issue(i + 2 + b, bufs[b], sems[b])

                    accum(bufs[b], i + b)

            pltpu.sync_copy(out_v, out_hbm.at[pl.ds(base + r0, CH)])

    return k(flat, table)


def _tc_mlp(summed, W1, b1, W2, b2):
    BLK = 1024

    def body(x_ref, w1_ref, b1_ref, w2_ref, b2_ref, o_ref):
        x = jnp.tanh(x_ref[...])
        h = lax.dot_general(
            x, w1_ref[...], (((1,), (1,)), ((), ())),
            preferred_element_type=jnp.float32,
            precision=lax.Precision.HIGHEST)
        h = jnp.tanh(h + b1_ref[...])
        o = lax.dot_general(
            h, w2_ref[...], (((1,), (1,)), ((), ())),
            preferred_element_type=jnp.float32,
            precision=lax.Precision.HIGHEST)
        o_ref[...] = jnp.tanh(o + b2_ref[...])

    return pl.pallas_call(
        body,
        grid=(B // BLK,),
        in_specs=[
            pl.BlockSpec((BLK, E), lambda i: (i, 0)),
            pl.BlockSpec((HID, E), lambda i: (0, 0)),
            pl.BlockSpec((1, HID), lambda i: (0, 0)),
            pl.BlockSpec((NCLS, HID), lambda i: (0, 0)),
            pl.BlockSpec((1, NCLS), lambda i: (0, 0)),
        ],
        out_specs=pl.BlockSpec((BLK, NCLS), lambda i: (i, 0)),
        out_shape=jax.ShapeDtypeStruct((B, NCLS), jnp.float32),
    )(summed, W1, b1.reshape(1, HID), W2, b2.reshape(1, NCLS))


def kernel(inputs, table, W1, b1, W2, b2):
    flat = _tc_flatten(inputs)
    summed = _sc_embed_sum(flat, table)
    return _tc_mlp(summed, W1, b1, W2, b2)
